# baseline (speedup 1.0000x reference)
import jax
import jax.numpy as jnp
from jax.experimental import pallas as pl
from jax.experimental.pallas import tpu as pltpu

_B, _H, _W = 32, 512, 512
_NC = 10
_BB = 4
_STEPS = _B // _BB
_CH = 64


def _kernel(o_ref, t_ref, m_ref, loss_ref, l4_ref, cn_ref, psA, pcA):
    j = pl.program_id(0)

    @pl.when(j == 0)
    def _():
        psA[...] = jnp.zeros((16, 8, _W), jnp.float32)
        pcA[...] = jnp.zeros((16, 8, _W), jnp.float32)

    for b in range(_BB):
        for hg in range(_H // _CH):
            oc = o_ref[b, hg * _CH:(hg + 1) * _CH, :]
            tc = t_ref[b, hg * _CH:(hg + 1) * _CH, :]
            mc = m_ref[b, hg * _CH:(hg + 1) * _CH, :]
            tmc = jnp.where(mc == 1, tc, _NC)
            dd = oc - tc.astype(jnp.float32)
            d2c = dd * dd
            for c in range(_NC):
                ef = jnp.where(tmc == c, 1.0, 0.0)
                psA[c] += jnp.sum((ef * d2c).reshape(_CH // 8, 8, _W), axis=0)
                pcA[c] += jnp.sum(ef.reshape(_CH // 8, 8, _W), axis=0)

    @pl.when(j == _STEPS - 1)
    def _():
        ps16 = jnp.sum(psA[...], axis=1)
        pc16 = jnp.sum(pcA[...], axis=1)
        s = jnp.sum(ps16, axis=1, keepdims=True)
        n = jnp.sum(pc16, axis=1, keepdims=True)
        l4 = jnp.where(n > 0, s / jnp.maximum(n, 1.0), 0.0)
        l4_b = jnp.broadcast_to(l4, (16, 128))
        n_b = jnp.broadcast_to(n, (16, 128))
        l4_ref[...] = l4_b
        cn_ref[...] = n_b
        loss_ref[...] = 0.1 * jnp.sum(l4_b, axis=0, keepdims=True)


def kernel(outputs, targets, mask):
    blk = pl.BlockSpec((_BB, _H, _W), lambda j: (j, 0, 0))
    out = pl.BlockSpec((1, 128), lambda j: (0, 0))
    out16 = pl.BlockSpec((16, 128), lambda j: (0, 0))

    loss_m, l4_m, cn_m = pl.pallas_call(
        _kernel,
        grid=(_STEPS,),
        in_specs=[blk, blk, blk],
        out_specs=[out, out16, out16],
        out_shape=[
            jax.ShapeDtypeStruct((1, 128), jnp.float32),
            jax.ShapeDtypeStruct((16, 128), jnp.float32),
            jax.ShapeDtypeStruct((16, 128), jnp.float32),
        ],
        scratch_shapes=[
            pltpu.VMEM((16, 8, _W), jnp.float32),
            pltpu.VMEM((16, 8, _W), jnp.float32),
        ],
        compiler_params=pltpu.CompilerParams(
            dimension_semantics=("arbitrary",),
        ),
    )(outputs, targets, mask)

    loss = loss_m[0, 0]
    loss4each = l4_m[:_NC, 0]
    class_n = cn_m[:_NC, 0]
    return loss, loss4each, class_n


# speedup vs baseline: 1.0241x; 1.0241x over previous
import jax
import jax.numpy as jnp
from jax.experimental import pallas as pl
from jax.experimental.pallas import tpu as pltpu

_B, _H, _W = 32, 512, 512
_NC = 10
_BB = 2
_STEPS = _B // _BB
_CH = 64


def _kernel(o_ref, t_ref, m_ref, loss_ref, l4_ref, cn_ref, psA, pcA):
    j = pl.program_id(0)

    @pl.when(j == 0)
    def _():
        psA[...] = jnp.zeros((16, 8, _W), jnp.float32)
        pcA[...] = jnp.zeros((16, 8, _W), jnp.float32)

    for b in range(_BB):
        for hg in range(_H // _CH):
            oc = o_ref[b, hg * _CH:(hg + 1) * _CH, :]
            tc = t_ref[b, hg * _CH:(hg + 1) * _CH, :]
            mc = m_ref[b, hg * _CH:(hg + 1) * _CH, :]
            tmc = jnp.where(mc == 1, tc, _NC)
            dd = oc - tc.astype(jnp.float32)
            d2c = dd * dd
            for c in range(_NC):
                ef = jnp.where(tmc == c, 1.0, 0.0)
                psA[c] += jnp.sum((ef * d2c).reshape(_CH // 8, 8, _W), axis=0)
                pcA[c] += jnp.sum(ef.reshape(_CH // 8, 8, _W), axis=0)

    @pl.when(j == _STEPS - 1)
    def _():
        ps16 = jnp.sum(psA[...], axis=1)
        pc16 = jnp.sum(pcA[...], axis=1)
        s = jnp.sum(ps16, axis=1, keepdims=True)
        n = jnp.sum(pc16, axis=1, keepdims=True)
        l4 = jnp.where(n > 0, s / jnp.maximum(n, 1.0), 0.0)
        l4_b = jnp.broadcast_to(l4, (16, 128))
        n_b = jnp.broadcast_to(n, (16, 128))
        l4_ref[...] = l4_b
        cn_ref[...] = n_b
        loss_ref[...] = 0.1 * jnp.sum(l4_b, axis=0, keepdims=True)


def kernel(outputs, targets, mask):
    blk = pl.BlockSpec((_BB, _H, _W), lambda j: (j, 0, 0))
    out = pl.BlockSpec((1, 128), lambda j: (0, 0))
    out16 = pl.BlockSpec((16, 128), lambda j: (0, 0))

    loss_m, l4_m, cn_m = pl.pallas_call(
        _kernel,
        grid=(_STEPS,),
        in_specs=[blk, blk, blk],
        out_specs=[out, out16, out16],
        out_shape=[
            jax.ShapeDtypeStruct((1, 128), jnp.float32),
            jax.ShapeDtypeStruct((16, 128), jnp.float32),
            jax.ShapeDtypeStruct((16, 128), jnp.float32),
        ],
        scratch_shapes=[
            pltpu.VMEM((16, 8, _W), jnp.float32),
            pltpu.VMEM((16, 8, _W), jnp.float32),
        ],
        compiler_params=pltpu.CompilerParams(
            dimension_semantics=("arbitrary",),
        ),
    )(outputs, targets, mask)

    loss = loss_m[0, 0]
    loss4each = l4_m[:_NC, 0]
    class_n = cn_m[:_NC, 0]
    return loss, loss4each, class_n
